# baseline (speedup 1.0000x reference)
import functools

import jax
import jax.numpy as jnp
from jax import lax
from jax.experimental import pallas as pl
from jax.experimental.pallas import tpu as pltpu
from jax.experimental.pallas import tpu_sc as plsc

VOCAB_SIZE = 1000000
EMB_SIZE = 32
BATCH = 16384
HIST = 50

NUM_CORES = 2
NUM_SUBCORES = 16
NUM_WORKERS = NUM_CORES * NUM_SUBCORES

TOTAL = BATCH * HIST
PER_WORKER = TOTAL // NUM_WORKERS
CHUNK = 1600
N_CHUNKS = PER_WORKER // CHUNK


def _gather_kernel(idx_hbm, table_hbm, out_hbm,
                   idx_v0, idx_v1, rows_v0, rows_v1,
                   gsem0, gsem1, osem0, osem1):
    wid = lax.axis_index("s") * NUM_CORES + lax.axis_index("c")
    base = wid * PER_WORKER

    idx_v = (idx_v0, idx_v1)
    rows_v = (rows_v0, rows_v1)
    gsem = (gsem0, gsem1)
    osem = (osem0, osem1)

    def start_gather(chunk_start, b):
        pltpu.sync_copy(idx_hbm.at[pl.ds(chunk_start, CHUNK)], idx_v[b])
        pltpu.async_copy(table_hbm.at[idx_v[b]], rows_v[b], gsem[b])

    def wait_gather(b):
        pltpu.make_async_copy(table_hbm.at[idx_v[b]], rows_v[b], gsem[b]).wait()

    def start_store(chunk_start, b):
        pltpu.async_copy(rows_v[b], out_hbm.at[pl.ds(chunk_start, CHUNK)],
                         osem[b])

    def wait_store(chunk_start, b):
        pltpu.make_async_copy(rows_v[b],
                              out_hbm.at[pl.ds(chunk_start, CHUNK)],
                              osem[b]).wait()

    start_gather(base, 0)
    start_gather(base + CHUNK, 1)

    @pl.loop(2, N_CHUNKS, step=2)
    def _(g):
        cur = base + g * CHUNK
        wait_gather(0)
        start_store(cur - 2 * CHUNK, 0)
        wait_store(cur - 2 * CHUNK, 0)
        start_gather(cur, 0)

        wait_gather(1)
        start_store(cur - CHUNK, 1)
        wait_store(cur - CHUNK, 1)
        start_gather(cur + CHUNK, 1)

    tail = base + (N_CHUNKS - 2) * CHUNK
    wait_gather(0)
    start_store(tail, 0)
    wait_gather(1)
    start_store(tail + CHUNK, 1)
    wait_store(tail, 0)
    wait_store(tail + CHUNK, 1)


@jax.jit
def _gather(idx_flat, table):
    mesh = plsc.VectorSubcoreMesh(core_axis_name="c", subcore_axis_name="s")
    run = functools.partial(
        pl.kernel,
        mesh=mesh,
        out_type=jax.ShapeDtypeStruct((TOTAL, EMB_SIZE), jnp.float32),
        scratch_types=[
            pltpu.VMEM((CHUNK,), jnp.int32),
            pltpu.VMEM((CHUNK,), jnp.int32),
            pltpu.VMEM((CHUNK, EMB_SIZE), jnp.float32),
            pltpu.VMEM((CHUNK, EMB_SIZE), jnp.float32),
            pltpu.SemaphoreType.DMA,
            pltpu.SemaphoreType.DMA,
            pltpu.SemaphoreType.DMA,
            pltpu.SemaphoreType.DMA,
        ],
        compiler_params=pltpu.CompilerParams(use_tc_tiling_on_sc=False),
    )(_gather_kernel)
    return run(idx_flat, table)


BSTRIP = 2048
NSTRIP = BATCH // BSTRIP
PACK = 128 // EMB_SIZE


def _tc_transpose_kernel(x_ref, o_ref):
    x = x_ref[...]
    y = x.reshape(BSTRIP // PACK, PACK, EMB_SIZE)
    z = jnp.transpose(y, (2, 0, 1))
    o_ref[...] = z.reshape(EMB_SIZE, BSTRIP)


@jax.jit
def _tc_transpose(x):
    return pl.pallas_call(
        _tc_transpose_kernel,
        grid=(HIST, NSTRIP),
        in_specs=[pl.BlockSpec((BSTRIP // PACK, 128),
                               lambda l, c: (l * NSTRIP + c, 0))],
        out_specs=pl.BlockSpec((EMB_SIZE, BSTRIP), lambda l, c: (l, c)),
        out_shape=jax.ShapeDtypeStruct((HIST * EMB_SIZE, BATCH), jnp.float32),
    )(x)


def kernel(indices, table):
    idx_t = jnp.transpose(indices.astype(jnp.int32), (1, 0))
    rows = _gather(idx_t.reshape(TOTAL), table)
    packed = rows.reshape(TOTAL // PACK, 128)
    out2 = _tc_transpose(packed)
    out3 = out2.reshape(HIST, EMB_SIZE, BATCH)
    return jnp.transpose(out3, (2, 0, 1))


# speedup vs baseline: 1.8079x; 1.8079x over previous
import functools

import jax
import jax.numpy as jnp
from jax import lax
from jax.experimental import pallas as pl
from jax.experimental.pallas import tpu as pltpu
from jax.experimental.pallas import tpu_sc as plsc

VOCAB_SIZE = 1000000
EMB_SIZE = 32
BATCH = 16384
HIST = 50

NUM_CORES = 2
NUM_SUBCORES = 16
NUM_WORKERS = NUM_CORES * NUM_SUBCORES

TOTAL = BATCH * HIST
PER_WORKER = TOTAL // NUM_WORKERS
CHUNK = 1600
N_CHUNKS = PER_WORKER // CHUNK


def _gather_kernel(idx_hbm, table_hbm, out_hbm,
                   idx_v0, idx_v1, rows_v0, rows_v1,
                   gsem0, gsem1, osem0, osem1):
    wid = lax.axis_index("s") * NUM_CORES + lax.axis_index("c")
    base = wid * PER_WORKER

    idx_v = (idx_v0, idx_v1)
    rows_v = (rows_v0, rows_v1)
    gsem = (gsem0, gsem1)
    osem = (osem0, osem1)

    def start_gather(chunk_start, b):
        pltpu.sync_copy(idx_hbm.at[pl.ds(chunk_start, CHUNK)], idx_v[b])
        pltpu.async_copy(table_hbm.at[idx_v[b]], rows_v[b], gsem[b])

    def wait_gather(b):
        pltpu.make_async_copy(table_hbm.at[idx_v[b]], rows_v[b], gsem[b]).wait()

    def start_store(chunk_start, b):
        pltpu.async_copy(rows_v[b], out_hbm.at[pl.ds(chunk_start, CHUNK)],
                         osem[b])

    def wait_store(chunk_start, b):
        pltpu.make_async_copy(rows_v[b],
                              out_hbm.at[pl.ds(chunk_start, CHUNK)],
                              osem[b]).wait()

    start_gather(base, 0)
    start_gather(base + CHUNK, 1)

    @pl.loop(2, N_CHUNKS, step=2)
    def _(g):
        cur = base + g * CHUNK
        wait_gather(0)
        start_store(cur - 2 * CHUNK, 0)
        wait_store(cur - 2 * CHUNK, 0)
        start_gather(cur, 0)

        wait_gather(1)
        start_store(cur - CHUNK, 1)
        wait_store(cur - CHUNK, 1)
        start_gather(cur + CHUNK, 1)

    tail = base + (N_CHUNKS - 2) * CHUNK
    wait_gather(0)
    start_store(tail, 0)
    wait_gather(1)
    start_store(tail + CHUNK, 1)
    wait_store(tail, 0)
    wait_store(tail + CHUNK, 1)


@jax.jit
def _gather(idx_flat, table):
    mesh = plsc.VectorSubcoreMesh(core_axis_name="c", subcore_axis_name="s")
    run = functools.partial(
        pl.kernel,
        mesh=mesh,
        out_type=jax.ShapeDtypeStruct((TOTAL, EMB_SIZE), jnp.float32),
        scratch_types=[
            pltpu.VMEM((CHUNK,), jnp.int32),
            pltpu.VMEM((CHUNK,), jnp.int32),
            pltpu.VMEM((CHUNK, EMB_SIZE), jnp.float32),
            pltpu.VMEM((CHUNK, EMB_SIZE), jnp.float32),
            pltpu.SemaphoreType.DMA,
            pltpu.SemaphoreType.DMA,
            pltpu.SemaphoreType.DMA,
            pltpu.SemaphoreType.DMA,
        ],
        compiler_params=pltpu.CompilerParams(use_tc_tiling_on_sc=False),
    )(_gather_kernel)
    return run(idx_flat, table)


BSTRIP = 2048
NSTRIP = BATCH // BSTRIP
PACK = 128 // EMB_SIZE


def _tc_transpose_kernel(x_ref, o_ref):
    x2 = x_ref[...]
    e_i = lax.broadcasted_iota(jnp.int32, (EMB_SIZE, EMB_SIZE), 0)
    e_j = lax.broadcasted_iota(jnp.int32, (EMB_SIZE, EMB_SIZE), 1)
    ident = jnp.where(e_i == e_j, jnp.float32(1), jnp.float32(0))
    o_ref[...] = lax.dot_general(ident, x2, (((1,), (1,)), ((), ())),
                                 preferred_element_type=jnp.float32,
                                 precision=lax.Precision.HIGHEST)


@jax.jit
def _tc_transpose(x):
    return pl.pallas_call(
        _tc_transpose_kernel,
        grid=(HIST, NSTRIP),
        in_specs=[pl.BlockSpec((BSTRIP, EMB_SIZE),
                               lambda l, c: (l * NSTRIP + c, 0))],
        out_specs=pl.BlockSpec((EMB_SIZE, BSTRIP), lambda l, c: (l, c)),
        out_shape=jax.ShapeDtypeStruct((HIST * EMB_SIZE, BATCH), jnp.float32),
    )(x)


def kernel(indices, table):
    idx_t = jnp.transpose(indices.astype(jnp.int32), (1, 0))
    rows = _gather(idx_t.reshape(TOTAL), table)
    out2 = _tc_transpose(rows)
    out3 = out2.reshape(HIST, EMB_SIZE, BATCH)
    return jnp.transpose(out3, (2, 0, 1))


# speedup vs baseline: 2.3137x; 1.2797x over previous
import functools

import jax
import jax.numpy as jnp
from jax import lax
from jax.experimental import pallas as pl
from jax.experimental.pallas import tpu as pltpu
from jax.experimental.pallas import tpu_sc as plsc

VOCAB_SIZE = 1000000
EMB_SIZE = 32
BATCH = 16384
HIST = 50

NUM_CORES = 2
NUM_SUBCORES = 16
NUM_WORKERS = NUM_CORES * NUM_SUBCORES

TOTAL = BATCH * HIST
PER_WORKER = TOTAL // NUM_WORKERS
CHUNK = 1600
N_CHUNKS = PER_WORKER // CHUNK


def _gather_kernel(idx_hbm, table_hbm, out_hbm,
                   idx_v0, idx_v1, rows_v0, rows_v1,
                   gsem0, gsem1, osem0, osem1):
    wid = lax.axis_index("s") * NUM_CORES + lax.axis_index("c")
    base = wid * PER_WORKER

    idx_v = (idx_v0, idx_v1)
    rows_v = (rows_v0, rows_v1)
    gsem = (gsem0, gsem1)
    osem = (osem0, osem1)

    def start_gather(chunk_start, b):
        pltpu.sync_copy(idx_hbm.at[pl.ds(chunk_start, CHUNK)], idx_v[b])
        pltpu.async_copy(table_hbm.at[idx_v[b]], rows_v[b], gsem[b])

    def wait_gather(b):
        pltpu.make_async_copy(table_hbm.at[idx_v[b]], rows_v[b], gsem[b]).wait()

    def start_store(chunk_start, b):
        pltpu.async_copy(rows_v[b], out_hbm.at[pl.ds(chunk_start, CHUNK)],
                         osem[b])

    def wait_store(chunk_start, b):
        pltpu.make_async_copy(rows_v[b],
                              out_hbm.at[pl.ds(chunk_start, CHUNK)],
                              osem[b]).wait()

    start_gather(base, 0)
    start_gather(base + CHUNK, 1)

    @pl.loop(2, N_CHUNKS, step=2)
    def _(g):
        cur = base + g * CHUNK
        wait_gather(0)
        start_store(cur - 2 * CHUNK, 0)
        wait_store(cur - 2 * CHUNK, 0)
        start_gather(cur, 0)

        wait_gather(1)
        start_store(cur - CHUNK, 1)
        wait_store(cur - CHUNK, 1)
        start_gather(cur + CHUNK, 1)

    tail = base + (N_CHUNKS - 2) * CHUNK
    wait_gather(0)
    start_store(tail, 0)
    wait_gather(1)
    start_store(tail + CHUNK, 1)
    wait_store(tail, 0)
    wait_store(tail + CHUNK, 1)


@jax.jit
def _gather(idx_flat, table):
    mesh = plsc.VectorSubcoreMesh(core_axis_name="c", subcore_axis_name="s")
    run = functools.partial(
        pl.kernel,
        mesh=mesh,
        out_type=jax.ShapeDtypeStruct((TOTAL, EMB_SIZE), jnp.float32),
        scratch_types=[
            pltpu.VMEM((CHUNK,), jnp.int32),
            pltpu.VMEM((CHUNK,), jnp.int32),
            pltpu.VMEM((CHUNK, EMB_SIZE), jnp.float32),
            pltpu.VMEM((CHUNK, EMB_SIZE), jnp.float32),
            pltpu.SemaphoreType.DMA,
            pltpu.SemaphoreType.DMA,
            pltpu.SemaphoreType.DMA,
            pltpu.SemaphoreType.DMA,
        ],
        compiler_params=pltpu.CompilerParams(use_tc_tiling_on_sc=False),
    )(_gather_kernel)
    return run(idx_flat, table)


BSTRIP = 2048
NSTRIP = BATCH // BSTRIP
PACK = 128 // EMB_SIZE


def _tc_transpose_kernel(x_ref, o_ref):
    x = x_ref[...]
    e_i = lax.broadcasted_iota(jnp.int32, (EMB_SIZE, 128), 0)
    c_i = lax.broadcasted_iota(jnp.int32, (EMB_SIZE, 128), 1)
    for p in range(PACK):
        sel = jnp.where(c_i == p * EMB_SIZE + e_i, jnp.float32(1),
                        jnp.float32(0))
        zp = lax.dot_general(sel, x, (((1,), (1,)), ((), ())),
                             preferred_element_type=jnp.float32,
                             precision=lax.Precision.HIGHEST)
        o_ref[:, p * (BSTRIP // PACK):(p + 1) * (BSTRIP // PACK)] = zp


@jax.jit
def _tc_transpose(x):
    return pl.pallas_call(
        _tc_transpose_kernel,
        grid=(HIST, NSTRIP),
        in_specs=[pl.BlockSpec((BSTRIP // PACK, 128),
                               lambda l, c: (l * NSTRIP + c, 0))],
        out_specs=pl.BlockSpec((EMB_SIZE, BSTRIP), lambda l, c: (l, c)),
        out_shape=jax.ShapeDtypeStruct((HIST * EMB_SIZE, BATCH), jnp.float32),
    )(x)


def kernel(indices, table):
    idx_t = jnp.transpose(indices.astype(jnp.int32), (1, 0))
    idx_perm = (idx_t.reshape(TOTAL // BSTRIP, PACK, BSTRIP // PACK)
                .transpose(0, 2, 1).reshape(TOTAL))
    rows = _gather(idx_perm, table)
    packed = rows.reshape(TOTAL // PACK, 128)
    out2 = _tc_transpose(packed)
    out3 = out2.reshape(HIST, EMB_SIZE, BATCH)
    return jnp.transpose(out3, (2, 0, 1))


# speedup vs baseline: 3.3903x; 1.4653x over previous
import functools

import jax
import jax.numpy as jnp
from jax import lax
from jax.experimental import pallas as pl
from jax.experimental.pallas import tpu as pltpu
from jax.experimental.pallas import tpu_sc as plsc

VOCAB_SIZE = 1000000
EMB_SIZE = 32
BATCH = 16384
HIST = 50

NUM_CORES = 2
NUM_SUBCORES = 16
NUM_WORKERS = NUM_CORES * NUM_SUBCORES

TOTAL = BATCH * HIST
PER_WORKER = TOTAL // NUM_WORKERS
CHUNK = 1600
N_CHUNKS = PER_WORKER // CHUNK


def _gather_kernel(idx_hbm, table_hbm, out_hbm,
                   idx_v0, idx_v1, rows_v0, rows_v1,
                   gsem0, gsem1, osem0, osem1):
    wid = lax.axis_index("s") * NUM_CORES + lax.axis_index("c")
    base = wid * PER_WORKER

    idx_v = (idx_v0, idx_v1)
    rows_v = (rows_v0, rows_v1)
    gsem = (gsem0, gsem1)
    osem = (osem0, osem1)

    def start_gather(chunk_start, b):
        pltpu.sync_copy(idx_hbm.at[pl.ds(chunk_start, CHUNK)], idx_v[b])
        pltpu.async_copy(table_hbm.at[idx_v[b]], rows_v[b], gsem[b])

    def wait_gather(b):
        pltpu.make_async_copy(table_hbm.at[idx_v[b]], rows_v[b], gsem[b]).wait()

    def start_store(chunk_start, b):
        pltpu.async_copy(rows_v[b], out_hbm.at[pl.ds(chunk_start, CHUNK)],
                         osem[b])

    def wait_store(chunk_start, b):
        pltpu.make_async_copy(rows_v[b],
                              out_hbm.at[pl.ds(chunk_start, CHUNK)],
                              osem[b]).wait()

    start_gather(base, 0)
    start_gather(base + CHUNK, 1)

    @pl.loop(2, N_CHUNKS, step=2)
    def _(g):
        cur = base + g * CHUNK
        wait_gather(0)
        start_store(cur - 2 * CHUNK, 0)
        wait_store(cur - 2 * CHUNK, 0)
        start_gather(cur, 0)

        wait_gather(1)
        start_store(cur - CHUNK, 1)
        wait_store(cur - CHUNK, 1)
        start_gather(cur + CHUNK, 1)

    tail = base + (N_CHUNKS - 2) * CHUNK
    wait_gather(0)
    start_store(tail, 0)
    wait_gather(1)
    start_store(tail + CHUNK, 1)
    wait_store(tail, 0)
    wait_store(tail + CHUNK, 1)


@jax.jit
def _gather(idx_flat, table):
    mesh = plsc.VectorSubcoreMesh(core_axis_name="c", subcore_axis_name="s")
    run = functools.partial(
        pl.kernel,
        mesh=mesh,
        out_type=jax.ShapeDtypeStruct((TOTAL, EMB_SIZE), jnp.float32),
        scratch_types=[
            pltpu.VMEM((CHUNK,), jnp.int32),
            pltpu.VMEM((CHUNK,), jnp.int32),
            pltpu.VMEM((CHUNK, EMB_SIZE), jnp.float32),
            pltpu.VMEM((CHUNK, EMB_SIZE), jnp.float32),
            pltpu.SemaphoreType.DMA,
            pltpu.SemaphoreType.DMA,
            pltpu.SemaphoreType.DMA,
            pltpu.SemaphoreType.DMA,
        ],
        compiler_params=pltpu.CompilerParams(use_tc_tiling_on_sc=False),
    )(_gather_kernel)
    return run(idx_flat, table)


BSTRIP = 2048
NSTRIP = BATCH // BSTRIP
PACK = 128 // EMB_SIZE


def _tc_transpose_kernel(x_ref, o_ref):
    x = x_ref[...]
    e_i = lax.broadcasted_iota(jnp.int32, (128, 128), 0)
    c_i = lax.broadcasted_iota(jnp.int32, (128, 128), 1)
    ident = jnp.where(e_i == c_i, jnp.float32(1), jnp.float32(0))
    xt = lax.dot_general(ident, x, (((1,), (1,)), ((), ())),
                         preferred_element_type=jnp.float32,
                         precision=lax.Precision.HIGHEST)
    for p in range(PACK):
        o_ref[:, p * (BATCH // PACK):(p + 1) * (BATCH // PACK)] = (
            xt[p * EMB_SIZE:(p + 1) * EMB_SIZE, :])


@jax.jit
def _tc_transpose(x):
    return pl.pallas_call(
        _tc_transpose_kernel,
        grid=(HIST,),
        in_specs=[pl.BlockSpec((BATCH // PACK, 128), lambda l: (l, 0))],
        out_specs=pl.BlockSpec((EMB_SIZE, BATCH), lambda l: (l, 0)),
        out_shape=jax.ShapeDtypeStruct((HIST * EMB_SIZE, BATCH), jnp.float32),
    )(x)


def kernel(indices, table):
    idx_t = jnp.transpose(indices.astype(jnp.int32), (1, 0))
    idx_perm = (idx_t.reshape(HIST, PACK, BATCH // PACK)
                .transpose(0, 2, 1).reshape(TOTAL))
    rows = _gather(idx_perm, table)
    packed = rows.reshape(TOTAL // PACK, 128)
    out2 = _tc_transpose(packed)
    out3 = out2.reshape(HIST, EMB_SIZE, BATCH)
    return jnp.transpose(out3, (2, 0, 1))
